# baseline (speedup 1.0000x reference)
import functools

import jax
import jax.numpy as jnp
from jax import lax
from jax.experimental import pallas as pl
from jax.experimental.pallas import tpu as pltpu, tpu_sc as plsc

_LANES = 16
_NW = 32


def _rsqrt(a):
    i = plsc.bitcast(a, jnp.int32)
    i = jnp.int32(0x5F3759DF) - (i >> 1)
    y = plsc.bitcast(i, jnp.float32)
    for _ in range(3):
        y = y * (1.5 - 0.5 * a * y * y)
    return y


def _body(seq, hid, ids_hbm, maskf_hbm, vals_hbm, table_hbm, pos_hbm,
          gamma_hbm, beta_hbm, out_hbm,
          ids_v, maskf_v, vals_v, rows_v, pos_v, gamma_v, beta_v,
          sg0, sg1, so0, so1):
    per_w = seq // _NW
    nchunk = per_w // _LANES
    wid = lax.axis_index("s") * 2 + lax.axis_index("c")
    base = wid * per_w

    pltpu.sync_copy(ids_hbm.at[0, pl.ds(base, per_w)], ids_v)
    pltpu.sync_copy(maskf_hbm.at[0, pl.ds(base, per_w)], maskf_v)
    pltpu.sync_copy(vals_hbm.at[0, pl.ds(base, per_w)], vals_v)
    pltpu.sync_copy(gamma_hbm, gamma_v)
    pltpu.sync_copy(beta_hbm, beta_v)

    sg = (sg0, sg1)
    so = (so0, so1)

    def in_descs(c, b):
        return (
            pltpu.make_async_copy(
                table_hbm.at[ids_v.at[pl.ds(c * _LANES, _LANES)]],
                rows_v.at[b], sg[b]),
            pltpu.make_async_copy(
                pos_hbm.at[0, pl.ds(base + c * _LANES, _LANES)],
                pos_v.at[b], sg[b]),
        )

    def out_desc(c, b):
        return pltpu.make_async_copy(
            rows_v.at[b],
            out_hbm.at[0, pl.ds(base + c * _LANES, _LANES)], so[b])

    rowi = lax.iota(jnp.int32, _LANES)
    zeros = jnp.zeros((_LANES,), jnp.float32)
    zero_i = jnp.zeros((_LANES,), jnp.int32)
    inv_h = jnp.float32(1.0 / hid)

    def compute(c, b):
        rb = rows_v.at[b]
        pb = pos_v.at[b]
        m = maskf_v[pl.ds(c * _LANES, _LANES)]
        v = vals_v[pl.ds(c * _LANES, _LANES)]
        scale = m * v - m + 1.0

        U = 4

        def body1(i, carry):
            s = list(carry[:U])
            q = list(carry[U:2 * U])
            hv = carry[2 * U]
            for j in range(U):
                hj = hv + j if j else hv
                x = plsc.load_gather(rb, [rowi, hj])
                p = plsc.load_gather(pb, [rowi, hj])
                t = x * scale + p
                plsc.store_scatter(rb, [rowi, hj], t)
                s[j] = s[j] + t
                q[j] = q[j] + t * t
            return (*s, *q, hv + U)

        res = plsc.parallel_loop(
            0, hid, step=U, carry=(zeros,) * U + (zeros,) * U + (zero_i,))(
                body1)
        s = (res[0] + res[1]) + (res[2] + res[3])
        q = (res[U] + res[U + 1]) + (res[U + 2] + res[U + 3])
        mean = s * inv_h
        var = q * inv_h - mean * mean
        rstd = _rsqrt(var + 1e-5)
        nm = -mean

        def body2(i, hv):
            for j in range(U):
                hj = hv + j if j else hv
                t = plsc.load_gather(rb, [rowi, hj])
                g = plsc.load_gather(gamma_v, [hj])
                bb = plsc.load_gather(beta_v, [hj])
                o = (t + nm) * (rstd * g) + bb
                plsc.store_scatter(rb, [rowi, hj], o)
            return hv + U

        plsc.parallel_loop(0, hid, step=U, carry=zero_i)(body2)

    def steady_step(c, b):
        out_desc(c - 1, 1 - b).wait()
        for d in in_descs(c + 1, 1 - b):
            d.start()
        for d in in_descs(c, b):
            d.wait()
        compute(c, b)
        out_desc(c, b).start()

    for d in in_descs(0, 0):
        d.start()
    for d in in_descs(1, 1):
        d.start()
    for d in in_descs(0, 0):
        d.wait()
    compute(0, 0)
    out_desc(0, 0).start()

    def steady(k, carry):
        c = 2 * k + 1
        steady_step(c, 1)
        steady_step(c + 1, 0)
        return carry

    lax.fori_loop(0, (nchunk - 2) // 2, steady, 0)

    c_last = nchunk - 1
    for d in in_descs(c_last, 1):
        d.wait()
    compute(c_last, 1)
    out_desc(c_last, 1).start()
    out_desc(c_last - 1, 0).wait()
    out_desc(c_last, 1).wait()


@functools.partial(jax.jit, static_argnames=())
def _sc_fused(ids, maskf, vals, table, pos, gamma, beta):
    seq = ids.shape[1]
    hid = table.shape[1]
    per_w = seq // _NW
    body = functools.partial(_body, seq, hid)
    return pl.kernel(
        body,
        out_type=jax.ShapeDtypeStruct((1, seq, hid), jnp.float32),
        mesh=plsc.VectorSubcoreMesh(core_axis_name="c", subcore_axis_name="s"),
        compiler_params=pltpu.CompilerParams(
            use_tc_tiling_on_sc=True, needs_layout_passes=False),
        scratch_types=[
            pltpu.VMEM((per_w,), jnp.int32),
            pltpu.VMEM((per_w,), jnp.float32),
            pltpu.VMEM((per_w,), jnp.float32),
            pltpu.VMEM((2, _LANES, hid), jnp.float32),
            pltpu.VMEM((2, _LANES, hid), jnp.float32),
            pltpu.VMEM((hid,), jnp.float32),
            pltpu.VMEM((hid,), jnp.float32),
            pltpu.SemaphoreType.DMA,
            pltpu.SemaphoreType.DMA,
            pltpu.SemaphoreType.DMA,
            pltpu.SemaphoreType.DMA,
        ],
    )(ids, maskf, vals, table, pos, gamma, beta)


def kernel(input_ids, num_mask, num_values, word_embeddings,
           positional_encoding, ln_gamma, ln_beta):
    ids = input_ids.astype(jnp.int32)
    maskf = num_mask.astype(jnp.float32)
    vals = num_values.astype(jnp.float32)
    return _sc_fused(ids, maskf, vals, word_embeddings, positional_encoding,
                     ln_gamma, ln_beta)


# speedup vs baseline: 5.3301x; 5.3301x over previous
import functools

import jax
import jax.numpy as jnp
from jax import lax
from jax.experimental import pallas as pl
from jax.experimental.pallas import tpu as pltpu, tpu_sc as plsc

_LANES = 16
_NW = 32


def _gather_body(seq, hid, ids_hbm, table_hbm, out_hbm,
                 ids_v, rows_v, sg0, sg1, so0, so1):
    per_w = seq // _NW
    nchunk = per_w // _LANES
    wid = lax.axis_index("s") * 2 + lax.axis_index("c")
    base = wid * per_w

    pltpu.sync_copy(ids_hbm.at[0, pl.ds(base, per_w)], ids_v)

    sg = (sg0, sg1)
    so = (so0, so1)

    def in_desc(c, b):
        return pltpu.make_async_copy(
            table_hbm.at[ids_v.at[pl.ds(c * _LANES, _LANES)]],
            rows_v.at[b], sg[b])

    def out_desc(c, b):
        return pltpu.make_async_copy(
            rows_v.at[b],
            out_hbm.at[0, pl.ds(base + c * _LANES, _LANES)], so[b])

    def steady_step(c, b):
        out_desc(c - 1, 1 - b).wait()
        in_desc(c + 1, 1 - b).start()
        in_desc(c, b).wait()
        out_desc(c, b).start()

    in_desc(0, 0).start()
    in_desc(1, 1).start()
    in_desc(0, 0).wait()
    out_desc(0, 0).start()

    def steady(k, carry):
        c = 2 * k + 1
        steady_step(c, 1)
        steady_step(c + 1, 0)
        return carry

    lax.fori_loop(0, (nchunk - 2) // 2, steady, 0)

    c_last = nchunk - 1
    in_desc(c_last, 1).wait()
    out_desc(c_last, 1).start()
    out_desc(c_last - 1, 0).wait()
    out_desc(c_last, 1).wait()


def _sc_gather(ids, table):
    seq = ids.shape[1]
    hid = table.shape[1]
    per_w = seq // _NW
    body = functools.partial(_gather_body, seq, hid)
    return pl.kernel(
        body,
        out_type=jax.ShapeDtypeStruct((1, seq, hid), jnp.float32),
        mesh=plsc.VectorSubcoreMesh(core_axis_name="c", subcore_axis_name="s"),
        compiler_params=pltpu.CompilerParams(use_tc_tiling_on_sc=True),
        scratch_types=[
            pltpu.VMEM((per_w,), jnp.int32),
            pltpu.VMEM((2, _LANES, hid), jnp.float32),
            pltpu.SemaphoreType.DMA,
            pltpu.SemaphoreType.DMA,
            pltpu.SemaphoreType.DMA,
            pltpu.SemaphoreType.DMA,
        ],
    )(ids, table)


def _ln_body(htext_ref, mask_ref, vals_ref, pos_ref, gamma_ref, beta_ref,
             out_ref):
    x = htext_ref[0]
    p = pos_ref[0]
    m = mask_ref[0].astype(jnp.float32)
    v = vals_ref[0]
    scale = (m * v - m + 1.0)[:, None]
    t = x * scale + p
    mean = jnp.mean(t, axis=-1, keepdims=True)
    var = jnp.mean(jnp.square(t - mean), axis=-1, keepdims=True)
    normed = (t - mean) * lax.rsqrt(var + 1e-5)
    out_ref[0] = normed * gamma_ref[...] + beta_ref[...]


def _tc_ln(htext, mask, vals, pos, gamma, beta):
    _, seq, hid = htext.shape
    blk = 1024
    row_spec = pl.BlockSpec((1, blk, hid), lambda i: (0, i, 0))
    tok_spec = pl.BlockSpec((1, blk), lambda i: (0, i))
    vec_spec = pl.BlockSpec((hid,), lambda i: (0,))
    return pl.pallas_call(
        _ln_body,
        grid=(seq // blk,),
        in_specs=[row_spec, tok_spec, tok_spec, row_spec, vec_spec, vec_spec],
        out_specs=row_spec,
        out_shape=jax.ShapeDtypeStruct((1, seq, hid), jnp.float32),
    )(htext, mask, vals, pos, gamma, beta)


@jax.jit
def _fused(ids, mask, vals, table, pos, gamma, beta):
    htext = _sc_gather(ids, table)
    return _tc_ln(htext, mask, vals, pos, gamma, beta)


def kernel(input_ids, num_mask, num_values, word_embeddings,
           positional_encoding, ln_gamma, ln_beta):
    ids = input_ids.astype(jnp.int32)
    return _fused(ids, num_mask, num_values.astype(jnp.float32),
                  word_embeddings, positional_encoding, ln_gamma, ln_beta)


# speedup vs baseline: 5.3709x; 1.0077x over previous
import functools

import jax
import jax.numpy as jnp
from jax import lax
from jax.experimental import pallas as pl
from jax.experimental.pallas import tpu as pltpu, tpu_sc as plsc

_LANES = 16
_NW = 32


def _gather_body(seq, hid, ids_hbm, table_hbm, out_hbm,
                 ids_v, rows_v, sg0, sg1, so0, so1):
    per_w = seq // _NW
    nchunk = per_w // _LANES
    wid = lax.axis_index("s") * 2 + lax.axis_index("c")
    base = wid * per_w

    pltpu.sync_copy(ids_hbm.at[0, pl.ds(base, per_w)], ids_v)

    sg = (sg0, sg1)
    so = (so0, so1)

    def in_desc(c, b):
        return pltpu.make_async_copy(
            table_hbm.at[ids_v.at[pl.ds(c * _LANES, _LANES)]],
            rows_v.at[b], sg[b])

    def out_desc(c, b):
        return pltpu.make_async_copy(
            rows_v.at[b],
            out_hbm.at[0, pl.ds(base + c * _LANES, _LANES)], so[b])

    def steady_step(c, b):
        out_desc(c - 1, 1 - b).wait()
        in_desc(c + 1, 1 - b).start()
        in_desc(c, b).wait()
        out_desc(c, b).start()

    in_desc(0, 0).start()
    in_desc(1, 1).start()
    in_desc(0, 0).wait()
    out_desc(0, 0).start()

    def steady(k, carry):
        c = 2 * k + 1
        steady_step(c, 1)
        steady_step(c + 1, 0)
        return carry

    lax.fori_loop(0, (nchunk - 2) // 2, steady, 0)

    c_last = nchunk - 1
    in_desc(c_last, 1).wait()
    out_desc(c_last, 1).start()
    out_desc(c_last - 1, 0).wait()
    out_desc(c_last, 1).wait()


def _sc_gather(ids, table):
    seq = ids.shape[1]
    hid = table.shape[1]
    per_w = seq // _NW
    body = functools.partial(_gather_body, seq, hid)
    return pl.kernel(
        body,
        out_type=jax.ShapeDtypeStruct((1, seq, hid), jnp.float32),
        mesh=plsc.VectorSubcoreMesh(core_axis_name="c", subcore_axis_name="s"),
        compiler_params=pltpu.CompilerParams(use_tc_tiling_on_sc=True),
        scratch_types=[
            pltpu.VMEM((per_w,), jnp.int32),
            pltpu.VMEM((2, _LANES, hid), jnp.float32),
            pltpu.SemaphoreType.DMA,
            pltpu.SemaphoreType.DMA,
            pltpu.SemaphoreType.DMA,
            pltpu.SemaphoreType.DMA,
        ],
    )(ids, table)


def _ln_body(htext_ref, mask_ref, vals_ref, pos_ref, gamma_ref, beta_ref,
             out_ref):
    x = htext_ref[0]
    p = pos_ref[0]
    m = mask_ref[0].astype(jnp.float32)
    v = vals_ref[0]
    scale = (m * v - m + 1.0)[:, None]
    t = x * scale + p
    mean = jnp.mean(t, axis=-1, keepdims=True)
    var = jnp.mean(jnp.square(t - mean), axis=-1, keepdims=True)
    normed = (t - mean) * lax.rsqrt(var + 1e-5)
    out_ref[0] = normed * gamma_ref[...] + beta_ref[...]


def _tc_ln(htext, mask, vals, pos, gamma, beta):
    _, seq, hid = htext.shape
    blk = 2048
    row_spec = pl.BlockSpec((1, blk, hid), lambda i: (0, i, 0))
    tok_spec = pl.BlockSpec((1, blk), lambda i: (0, i))
    vec_spec = pl.BlockSpec((hid,), lambda i: (0,))
    return pl.pallas_call(
        _ln_body,
        grid=(seq // blk,),
        in_specs=[row_spec, tok_spec, tok_spec, row_spec, vec_spec, vec_spec],
        out_specs=row_spec,
        out_shape=jax.ShapeDtypeStruct((1, seq, hid), jnp.float32),
    )(htext, mask, vals, pos, gamma, beta)


@jax.jit
def _fused(ids, mask, vals, table, pos, gamma, beta):
    htext = _sc_gather(ids, table)
    return _tc_ln(htext, mask, vals, pos, gamma, beta)


def kernel(input_ids, num_mask, num_values, word_embeddings,
           positional_encoding, ln_gamma, ln_beta):
    ids = input_ids.astype(jnp.int32)
    return _fused(ids, num_mask, num_values.astype(jnp.float32),
                  word_embeddings, positional_encoding, ln_gamma, ln_beta)


# speedup vs baseline: 5.4716x; 1.0187x over previous
import functools

import jax
import jax.numpy as jnp
from jax import lax
from jax.experimental import pallas as pl
from jax.experimental.pallas import tpu as pltpu, tpu_sc as plsc

_LANES = 16
_NW = 32


_CH = 32


def _gather_body(seq, hid, ids_hbm, table_hbm, out_hbm,
                 ids_v, rows_v, sg0, sg1, so0, so1):
    per_w = seq // _NW
    nchunk = per_w // _CH
    wid = lax.axis_index("s") * 2 + lax.axis_index("c")
    base = wid * per_w

    pltpu.sync_copy(ids_hbm.at[0, pl.ds(base, per_w)], ids_v)

    sg = (sg0, sg1)
    so = (so0, so1)

    def in_desc(c, b):
        return pltpu.make_async_copy(
            table_hbm.at[ids_v.at[pl.ds(c * _CH, _CH)]],
            rows_v.at[b], sg[b])

    def out_desc(c, b):
        return pltpu.make_async_copy(
            rows_v.at[b],
            out_hbm.at[0, pl.ds(base + c * _CH, _CH)], so[b])

    def steady_step(c, b):
        out_desc(c - 1, 1 - b).wait()
        in_desc(c + 1, 1 - b).start()
        in_desc(c, b).wait()
        out_desc(c, b).start()

    in_desc(0, 0).start()
    in_desc(1, 1).start()
    in_desc(0, 0).wait()
    out_desc(0, 0).start()

    def steady(k, carry):
        c = 2 * k + 1
        steady_step(c, 1)
        steady_step(c + 1, 0)
        return carry

    lax.fori_loop(0, (nchunk - 2) // 2, steady, 0)

    c_last = nchunk - 1
    in_desc(c_last, 1).wait()
    out_desc(c_last, 1).start()
    out_desc(c_last - 1, 0).wait()
    out_desc(c_last, 1).wait()


def _sc_gather(ids, table):
    seq = ids.shape[1]
    hid = table.shape[1]
    per_w = seq // _NW
    body = functools.partial(_gather_body, seq, hid)
    return pl.kernel(
        body,
        out_type=jax.ShapeDtypeStruct((1, seq, hid), jnp.float32),
        mesh=plsc.VectorSubcoreMesh(core_axis_name="c", subcore_axis_name="s"),
        compiler_params=pltpu.CompilerParams(use_tc_tiling_on_sc=True),
        scratch_types=[
            pltpu.VMEM((per_w,), jnp.int32),
            pltpu.VMEM((2, _CH, hid), jnp.float32),
            pltpu.SemaphoreType.DMA,
            pltpu.SemaphoreType.DMA,
            pltpu.SemaphoreType.DMA,
            pltpu.SemaphoreType.DMA,
        ],
    )(ids, table)


def _ln_body(htext_ref, mask_ref, vals_ref, pos_ref, gamma_ref, beta_ref,
             out_ref):
    x = htext_ref[0]
    p = pos_ref[0]
    m = mask_ref[0].astype(jnp.float32)
    v = vals_ref[0]
    scale = (m * v - m + 1.0)[:, None]
    t = x * scale + p
    mean = jnp.mean(t, axis=-1, keepdims=True)
    var = jnp.mean(jnp.square(t - mean), axis=-1, keepdims=True)
    normed = (t - mean) * lax.rsqrt(var + 1e-5)
    out_ref[0] = normed * gamma_ref[...] + beta_ref[...]


def _tc_ln(htext, mask, vals, pos, gamma, beta):
    _, seq, hid = htext.shape
    blk = 2048
    row_spec = pl.BlockSpec((1, blk, hid), lambda i: (0, i, 0))
    tok_spec = pl.BlockSpec((1, blk), lambda i: (0, i))
    vec_spec = pl.BlockSpec((hid,), lambda i: (0,))
    return pl.pallas_call(
        _ln_body,
        grid=(seq // blk,),
        in_specs=[row_spec, tok_spec, tok_spec, row_spec, vec_spec, vec_spec],
        out_specs=row_spec,
        out_shape=jax.ShapeDtypeStruct((1, seq, hid), jnp.float32),
    )(htext, mask, vals, pos, gamma, beta)


@jax.jit
def _fused(ids, mask, vals, table, pos, gamma, beta):
    htext = _sc_gather(ids, table)
    return _tc_ln(htext, mask, vals, pos, gamma, beta)


def kernel(input_ids, num_mask, num_values, word_embeddings,
           positional_encoding, ln_gamma, ln_beta):
    ids = input_ids.astype(jnp.int32)
    return _fused(ids, num_mask, num_values.astype(jnp.float32),
                  word_embeddings, positional_encoding, ln_gamma, ln_beta)
